# Initial kernel scaffold; baseline (speedup 1.0000x reference)
import jax
import jax.numpy as jnp
from jax.experimental import pallas as pl


def kernel(output, target, one_hot):
    raise NotImplementedError("write your pallas kernel here")


# speedup vs baseline: 2.3970x; 2.3970x over previous
import functools

import jax
import jax.numpy as jnp
from jax.experimental import pallas as pl
from jax.experimental.pallas import tpu as pltpu

_CONF = 0.9


def _body(nblk, B, V, Wb, out_ref, t_ref, oh_ref, res_ref,
          accw_ref, acck_ref, g_ref, oht_ref):
    k = pl.program_id(0)

    @pl.when(k == 0)
    def _init():
        accw_ref[0, 0] = 0.0
        acck_ref[0, 0] = 0.0
        g_ref[...] = jnp.zeros_like(g_ref)
        oht_ref[...] = jnp.zeros_like(oht_ref)

    x = out_ref[...]
    oh = oh_ref[...]
    col = jax.lax.broadcasted_iota(jnp.int32, (1, Wb), 1) + k * Wb
    valid = col < V

    colsum = jnp.sum(x, axis=0, keepdims=True)
    accw_ref[0, 0] += jnp.sum(jnp.where(valid, colsum * oh, 0.0))

    safe = jnp.where(oh > 0, oh, 1.0)
    kterm = jnp.where(valid & (oh > 0), oh * jnp.log(safe), 0.0)
    acck_ref[0, 0] += jnp.sum(kterm)

    tcol = t_ref[...]
    cols2 = jax.lax.broadcasted_iota(jnp.int32, (B, Wb), 1) + k * Wb
    mask = cols2 == tcol
    g_ref[...] += jnp.sum(jnp.where(mask, x, 0.0), axis=1, keepdims=True)
    ohb = jnp.broadcast_to(oh, (B, Wb))
    oht_ref[...] += jnp.sum(jnp.where(mask, ohb, 0.0), axis=1, keepdims=True)

    @pl.when(k == nblk - 1)
    def _fin():
        g = g_ref[...]
        oht = oht_ref[...]
        safe_t = jnp.where(oht > 0, oht, 1.0)
        xlogy_t = jnp.where(oht > 0, oht * jnp.log(safe_t), 0.0)
        corr = _CONF * jnp.log(_CONF) - _CONF * g - xlogy_t + oht * g
        res_ref[0, 0] = (B * acck_ref[0, 0] - accw_ref[0, 0] + jnp.sum(corr))


def kernel(output, target, one_hot):
    B, V = output.shape
    Wb = 2048
    nblk = pl.cdiv(V, Wb)

    t2 = target.reshape(B, 1)
    oh2 = one_hot.reshape(1, V)

    res = pl.pallas_call(
        functools.partial(_body, nblk, B, V, Wb),
        grid=(nblk,),
        in_specs=[
            pl.BlockSpec((B, Wb), lambda k: (0, k)),
            pl.BlockSpec((B, 1), lambda k: (0, 0)),
            pl.BlockSpec((1, Wb), lambda k: (0, k)),
        ],
        out_specs=pl.BlockSpec(memory_space=pltpu.SMEM),
        out_shape=jax.ShapeDtypeStruct((1, 1), jnp.float32),
        scratch_shapes=[
            pltpu.SMEM((1, 1), jnp.float32),
            pltpu.SMEM((1, 1), jnp.float32),
            pltpu.VMEM((B, 1), jnp.float32),
            pltpu.VMEM((B, 1), jnp.float32),
        ],
        compiler_params=pltpu.CompilerParams(
            dimension_semantics=("arbitrary",),
        ),
    )(output, t2, oh2)
    return res[0, 0]
